# baseline (speedup 1.0000x reference)
import functools

import jax
import jax.numpy as jnp
from jax import lax
from jax.experimental import pallas as pl
from jax.experimental.pallas import tpu as pltpu
from jax.experimental.pallas import tpu_sc as plsc

NG = 4096
NA = 100000
NE = 119
NS, L = 16, 16
ROW = 4104
GPT = NG // NS
APT = 6272
APT_LAST = NA - (NS - 1) * APT


@functools.cache
def _build():
  mesh = plsc.VectorSubcoreMesh(
      core_axis_name="c", subcore_axis_name="s",
      num_cores=1, num_subcores=NS)

  @functools.partial(
      pl.kernel,
      out_type=jax.ShapeDtypeStruct((NG,), jnp.float32),
      mesh=mesh,
      compiler_params=pltpu.CompilerParams(needs_layout_passes=False),
      scratch_types=[
          pltpu.VMEM((APT,), jnp.int32),
          pltpu.VMEM((APT,), jnp.int32),
          pltpu.VMEM((NE,), jnp.float32),
          pltpu.VMEM((ROW,), jnp.float32),
          pltpu.VMEM((NS, GPT), jnp.float32),
          pltpu.VMEM((GPT,), jnp.float32),
          pltpu.VMEM((GPT,), jnp.int32),
          pltpu.VMEM((1,), jnp.float32),
          pltpu.VMEM((1,), jnp.float32),
          pltpu.VMEM((GPT,), jnp.float32),
          pltpu.VMEM_SHARED((NS, ROW), jnp.float32),
          pltpu.SemaphoreType.DMA,
          pltpu.SemaphoreType.DMA,
      ],
  )
  def _fused(energy_hbm, natoms_hbm, z_hbm, img_hbm, scale_hbm, shift_hbm,
             ae_hbm, zrow_hbm, out_hbm,
             z_v, g_v, ae_v, acc_v, cmb_v, en_v, na_v, sc_v, sh_v, res_v,
             rows_sh, semA, semB):
    s = lax.axis_index("s")
    base = jnp.minimum(s * APT, NA - APT)
    g0 = s * GPT

    cp_ae = pltpu.async_copy(ae_hbm, ae_v, semA)
    cp_zero = pltpu.async_copy(zrow_hbm, acc_v, semA)
    cp_en = pltpu.async_copy(energy_hbm.at[pl.ds(g0, GPT)], en_v, semB)
    cp_na = pltpu.async_copy(natoms_hbm.at[pl.ds(g0, GPT)], na_v, semB)
    cp_sc = pltpu.async_copy(scale_hbm, sc_v, semB)
    cp_sh = pltpu.async_copy(shift_hbm, sh_v, semB)

    off = pl.multiple_of(base, 8)
    cp_z = pltpu.async_copy(z_hbm.at[pl.ds(off, APT)], z_v, semA)
    cp_g = pltpu.async_copy(img_hbm.at[pl.ds(off, APT)], g_v, semA)

    with jax.named_scope("ph_dma_in"):
        cp_ae.wait()
        cp_zero.wait()
        cp_z.wait()
        cp_g.wait()

    NV = APT // L
    iota = lax.iota(jnp.int32, L)
    lane_base = iota * NV
    D = NS * APT - NA
    m_pre = jnp.logical_or(iota != 0, jnp.full((L,), s < NS - 1))

    with jax.named_scope("ph_sweep"):
        i = 0
        while i < NV:
            g_n = min(8, NV - i)
            idxs = [lane_base + (i + j) for j in range(g_n)]
            zs = [plsc.load_gather(z_v, [ix]) for ix in idxs]
            gs = [plsc.load_gather(g_v, [ix]) for ix in idxs]
            vals = [plsc.load_gather(ae_v, [z]) for z in zs]
            for j, (g, v) in enumerate(zip(gs, vals)):
                if i + j < D:
                    plsc.addupdate_scatter(acc_v, [g], v, mask=m_pre)
                else:
                    plsc.addupdate_scatter(acc_v, [g], v)
            i += g_n

    with jax.named_scope("ph_stage"):
        pltpu.sync_copy(acc_v, rows_sh.at[s])
        plsc.subcore_barrier()
        pltpu.sync_copy(rows_sh.at[:, pl.ds(g0, GPT)], cmb_v)

    with jax.named_scope("ph_finish"):
        cp_en.wait()
        cp_na.wait()
        cp_sc.wait()
        cp_sh.wait()
        zero16 = jnp.zeros((L,), jnp.int32)
        scale = plsc.load_gather(sc_v, [zero16])
        shift = plsc.load_gather(sh_v, [zero16])
        for k in range(GPT // L):
            sl = pl.ds(k * L, L)
            acc = cmb_v[0, sl]
            for r in range(1, NS):
                acc = acc + cmb_v[r, sl]
            res_v[sl] = (en_v[sl] * scale
                         + na_v[sl].astype(jnp.float32) * shift + acc)
        pltpu.sync_copy(res_v, out_hbm.at[pl.ds(g0, GPT)])

  return _fused


def kernel(energy, n_atoms, Z, image_idx, scale_by, shift_by, atomic_energies):
    zrow = jnp.zeros((ROW,), jnp.float32)
    return _build()(
        energy, n_atoms.astype(jnp.int32), Z.astype(jnp.int32),
        image_idx.astype(jnp.int32), scale_by.astype(jnp.float32),
        shift_by.astype(jnp.float32), atomic_energies.astype(jnp.float32),
        zrow)
